# baseline (speedup 1.0000x reference)
import jax
import jax.numpy as jnp
from jax.experimental import pallas as pl


def _copy_body(x_ref, o_ref):
    o_ref[...] = x_ref[...]


def kernel(x, edge_index, edge_labels, edge_table, Ws, Us, w1, b1, w2, b2):
    src = edge_index[0]
    dst = edge_index[1]
    e_emb = jnp.take(edge_table, edge_labels, axis=0)
    h = x
    for l in range(3):
        msg = jnp.take(h, src, axis=0) + e_emb
        agg = jax.ops.segment_sum(msg, dst, num_segments=10000)
        h = jnp.tanh(h @ Ws[l] + agg @ Us[l])
    pooled = jnp.max(h[None, :, :], axis=1)
    logits = jnp.tanh(pooled @ w1 + b1) @ w2 + b2
    return pl.pallas_call(
        _copy_body, out_shape=jax.ShapeDtypeStruct(logits.shape, logits.dtype)
    )(logits)


# speedup vs baseline: 7.0777x; 7.0777x over previous
import functools

import jax
import jax.numpy as jnp
from jax import lax
from jax.experimental import pallas as pl
from jax.experimental.pallas import tpu as pltpu
from jax.experimental.pallas import tpu_sc as plsc

N_NODES = 10000
E_DIM = 128
N_EDGES = 320000
NUM_LAYER = 3

NC = 2
NS = 16
NW = NC * NS
EPW = N_EDGES // NW
CH = 100
NCHUNK = EPW // CH
ACC_ROWS = 10240
ROWS_PER_SUB = ACC_ROWS // NS


def _segsum_body(table_hbm, gidx_hbm, sidx_hbm, zero_hbm, out_hbm,
                 acc_shared, gidx_v, sidx_v, rows_v, sem):
    c = lax.axis_index("c")
    s = lax.axis_index("s")
    w = c * NS + s
    pltpu.sync_copy(zero_hbm.at[pl.ds(s * ROWS_PER_SUB, ROWS_PER_SUB)],
                    acc_shared.at[pl.ds(s * ROWS_PER_SUB, ROWS_PER_SUB)])
    pltpu.sync_copy(gidx_hbm.at[w], gidx_v)
    pltpu.sync_copy(sidx_hbm.at[w], sidx_v)
    plsc.subcore_barrier()

    def body(j, carry):
        pltpu.async_copy(table_hbm.at[gidx_v.at[j]], rows_v, sem).wait()
        pltpu.sync_copy(rows_v, acc_shared.at[sidx_v.at[j]], add=True)
        return carry

    lax.fori_loop(0, NCHUNK, body, 0)
    plsc.subcore_barrier()
    pltpu.sync_copy(acc_shared.at[pl.ds(s * ROWS_PER_SUB, ROWS_PER_SUB)],
                    out_hbm.at[c, pl.ds(s * ROWS_PER_SUB, ROWS_PER_SUB)])


def _segsum(table, gidx, sidx, zeros):
    mesh = plsc.VectorSubcoreMesh(core_axis_name="c", subcore_axis_name="s")
    f = pl.kernel(
        _segsum_body,
        out_type=jax.ShapeDtypeStruct((NC, ACC_ROWS, E_DIM), jnp.float32),
        mesh=mesh,
        scratch_types=[
            pltpu.VMEM_SHARED((ACC_ROWS, E_DIM), jnp.float32),
            pltpu.VMEM((NCHUNK, CH), jnp.int32),
            pltpu.VMEM((NCHUNK, CH), jnp.int32),
            pltpu.VMEM((CH, E_DIM), jnp.float32),
            pltpu.SemaphoreType.DMA,
        ],
    )
    return f(table, gidx, sidx, zeros)


def _layer_body(h_ref, s0_ref, s1_ref, e0_ref, e1_ref, w_ref, u_ref, o_ref):
    agg = s0_ref[...] + s1_ref[...] + e0_ref[...] + e1_ref[...]
    o_ref[...] = jnp.tanh(
        jnp.dot(h_ref[...], w_ref[...], preferred_element_type=jnp.float32)
        + jnp.dot(agg, u_ref[...], preferred_element_type=jnp.float32))


_BLK = 2000


def _layer(h, s0, s1, e0, e1, W, U):
    grid = (N_NODES // _BLK,)
    row_spec = pl.BlockSpec((_BLK, E_DIM), lambda i: (i, 0))
    mat_spec = pl.BlockSpec((E_DIM, E_DIM), lambda i: (0, 0))
    return pl.pallas_call(
        _layer_body,
        grid=grid,
        in_specs=[row_spec] * 5 + [mat_spec] * 2,
        out_specs=row_spec,
        out_shape=jax.ShapeDtypeStruct((N_NODES, E_DIM), jnp.float32),
    )(h, s0, s1, e0, e1, W, U)


def _final_body(h_ref, s0_ref, s1_ref, e0_ref, e1_ref, w_ref, u_ref,
                w1_ref, b1_ref, w2_ref, b2_ref, o_ref, mx_ref):
    i = pl.program_id(0)
    agg = s0_ref[...] + s1_ref[...] + e0_ref[...] + e1_ref[...]
    hb = jnp.tanh(
        jnp.dot(h_ref[...], w_ref[...], preferred_element_type=jnp.float32)
        + jnp.dot(agg, u_ref[...], preferred_element_type=jnp.float32))
    bmax = jnp.max(hb, axis=0, keepdims=True)

    @pl.when(i == 0)
    def _():
        mx_ref[...] = bmax

    @pl.when(i > 0)
    def _():
        mx_ref[...] = jnp.maximum(mx_ref[...], bmax)

    @pl.when(i == pl.num_programs(0) - 1)
    def _():
        pooled = mx_ref[...]
        hid = jnp.tanh(
            jnp.dot(pooled, w1_ref[...], preferred_element_type=jnp.float32)
            + b1_ref[...])
        o_ref[...] = (
            jnp.dot(hid, w2_ref[...], preferred_element_type=jnp.float32)
            + b2_ref[...])


def _final(h, s0, s1, e0, e1, W, U, w1p, b1p, w2p, b2p):
    grid = (N_NODES // _BLK,)
    row_spec = pl.BlockSpec((_BLK, E_DIM), lambda i: (i, 0))
    mat_spec = pl.BlockSpec((E_DIM, E_DIM), lambda i: (0, 0))
    vec_spec = pl.BlockSpec((1, E_DIM), lambda i: (0, 0))
    return pl.pallas_call(
        _final_body,
        grid=grid,
        in_specs=[row_spec] * 5 + [mat_spec] * 2
        + [mat_spec, vec_spec, mat_spec, vec_spec],
        out_specs=vec_spec,
        out_shape=jax.ShapeDtypeStruct((1, E_DIM), jnp.float32),
        scratch_shapes=[pltpu.VMEM((1, E_DIM), jnp.float32)],
    )(h, s0, s1, e0, e1, W, U, w1p, b1p, w2p, b2p)


def kernel(x, edge_index, edge_labels, edge_table, Ws, Us, w1, b1, w2, b2):
    src = edge_index[0].astype(jnp.int32).reshape(NW, NCHUNK, CH)
    dst = edge_index[1].astype(jnp.int32).reshape(NW, NCHUNK, CH)
    lab = edge_labels.astype(jnp.int32).reshape(NW, NCHUNK, CH)
    zeros = jnp.zeros((ACC_ROWS, E_DIM), jnp.float32)

    hid = w1.shape[1]
    nout = w2.shape[1]
    w1p = jnp.zeros((E_DIM, E_DIM), jnp.float32).at[:, :hid].set(w1)
    b1p = jnp.zeros((1, E_DIM), jnp.float32).at[0, :hid].set(b1)
    w2p = jnp.zeros((E_DIM, E_DIM), jnp.float32).at[:hid, :nout].set(w2)
    b2p = jnp.zeros((1, E_DIM), jnp.float32).at[0, :nout].set(b2)

    epart = _segsum(edge_table, lab, dst, zeros)[:, :N_NODES]

    h = x
    for l in range(NUM_LAYER):
        spart = _segsum(h, src, dst, zeros)[:, :N_NODES]
        if l < NUM_LAYER - 1:
            h = _layer(h, spart[0], spart[1], epart[0], epart[1], Ws[l], Us[l])
        else:
            out = _final(h, spart[0], spart[1], epart[0], epart[1],
                         Ws[l], Us[l], w1p, b1p, w2p, b2p)
    return out[:, :nout]
